# baseline (speedup 1.0000x reference)
import jax
import jax.numpy as jnp
import numpy as np
from jax.experimental import pallas as pl
from jax.experimental.pallas import tpu as pltpu

_ANCHORS = np.array(
    [[16, 8], [23, 103], [28, 23], [56, 47], [96, 123], [157, 248]],
    dtype=np.float32,
)
_NUM_CLASSES = 80
_IMG_DIM = 512.0
_NA = 6
_ATTRS = 5 + _NUM_CLASSES
_NG = 64
_NPOS = _NG * _NG
_STRIDE = _IMG_DIM / _NG


def _decode_kernel(x_ref, o_ref):
    a = pl.program_id(1)
    t = x_ref[0].reshape(_ATTRS, _NPOS)
    iota = jax.lax.broadcasted_iota(jnp.int32, (1, _NPOS), 1)
    gx = (iota % _NG).astype(jnp.float32)
    gy = (iota // _NG).astype(jnp.float32)
    r0 = (jax.nn.sigmoid(t[0:1, :]) + gx) * _STRIDE
    r1 = (jax.nn.sigmoid(t[1:2, :]) + gy) * _STRIDE
    aw = jnp.float32(_ANCHORS[_NA - 1, 0])
    ah = jnp.float32(_ANCHORS[_NA - 1, 1])
    for i in range(_NA - 1):
        aw = jnp.where(a == i, jnp.float32(_ANCHORS[i, 0]), aw)
        ah = jnp.where(a == i, jnp.float32(_ANCHORS[i, 1]), ah)
    r2 = jnp.exp(t[2:3, :]) * aw
    r3 = jnp.exp(t[3:4, :]) * ah
    r4 = jax.nn.sigmoid(t[4:5, :])
    full = jnp.concatenate([r0, r1, r2, r3, r4, t[5:, :]], axis=0)
    o_ref[0] = full.T


def kernel(x):
    nB = x.shape[0]
    return pl.pallas_call(
        _decode_kernel,
        grid=(nB, _NA),
        in_specs=[
            pl.BlockSpec((1, _ATTRS, _NG, _NG), lambda b, a: (b, a, 0, 0)),
        ],
        out_specs=pl.BlockSpec((1, _NPOS, _ATTRS), lambda b, a: (b, a, 0)),
        out_shape=jax.ShapeDtypeStruct((nB, _NA * _NPOS, _ATTRS), jnp.float32),
        compiler_params=pltpu.CompilerParams(
            dimension_semantics=("parallel", "parallel"),
        ),
    )(x)


# speedup vs baseline: 3.0141x; 3.0141x over previous
import jax
import jax.numpy as jnp
import numpy as np
from jax.experimental import pallas as pl
from jax.experimental.pallas import tpu as pltpu

_ANCHORS = np.array(
    [[16, 8], [23, 103], [28, 23], [56, 47], [96, 123], [157, 248]],
    dtype=np.float32,
)
_NUM_CLASSES = 80
_IMG_DIM = 512.0
_NA = 6
_ATTRS = 5 + _NUM_CLASSES
_NG = 64
_NPOS = _NG * _NG
_NCH = _NA * _ATTRS
_STRIDE = _IMG_DIM / _NG

_BB = 8
_YB = 8
_PB = _YB * _NG


def _decode_kernel(x_ref, o_ref, tr_ref):
    yc = pl.program_id(1)
    a = pl.program_id(2)

    @pl.when(a == 0)
    def _transpose_block():
        for bl in range(_BB):
            sub = x_ref[bl].reshape(_PB, _NCH)
            tr_ref[:, bl, :] = sub.T

    base = a * _ATTRS
    o_ref[:, :, :] = tr_ref[pl.ds(base, _ATTRS)]
    iota = jax.lax.broadcasted_iota(jnp.int32, (1, 1, _PB), 2)
    gx = (iota % _NG).astype(jnp.float32)
    gy = (yc * _YB + iota // _NG).astype(jnp.float32)
    aw = jnp.float32(_ANCHORS[_NA - 1, 0])
    ah = jnp.float32(_ANCHORS[_NA - 1, 1])
    for i in range(_NA - 1):
        aw = jnp.where(a == i, jnp.float32(_ANCHORS[i, 0]), aw)
        ah = jnp.where(a == i, jnp.float32(_ANCHORS[i, 1]), ah)
    r0 = tr_ref[pl.ds(base + 0, 1)]
    r1 = tr_ref[pl.ds(base + 1, 1)]
    r2 = tr_ref[pl.ds(base + 2, 1)]
    r3 = tr_ref[pl.ds(base + 3, 1)]
    r4 = tr_ref[pl.ds(base + 4, 1)]
    o_ref[0:1] = (jax.nn.sigmoid(r0) + gx) * _STRIDE
    o_ref[1:2] = (jax.nn.sigmoid(r1) + gy) * _STRIDE
    o_ref[2:3] = jnp.exp(r2) * aw
    o_ref[3:4] = jnp.exp(r3) * ah
    o_ref[4:5] = jax.nn.sigmoid(r4)


def kernel(x):
    nB = x.shape[0]
    xt = jnp.transpose(x, (0, 2, 3, 1))
    yt = pl.pallas_call(
        _decode_kernel,
        grid=(nB // _BB, _NG // _YB, _NA),
        in_specs=[
            pl.BlockSpec(
                (_BB, _YB, _NG, _NCH), lambda b8, yc, a: (b8, yc, 0, 0)
            ),
        ],
        out_specs=pl.BlockSpec(
            (_ATTRS, _BB, _PB), lambda b8, yc, a: (0, b8, a * (_NG // _YB) + yc)
        ),
        out_shape=jax.ShapeDtypeStruct((_ATTRS, nB, _NA * _NPOS), jnp.float32),
        scratch_shapes=[pltpu.VMEM((_NCH, _BB, _PB), jnp.float32)],
        compiler_params=pltpu.CompilerParams(
            dimension_semantics=("parallel", "arbitrary", "arbitrary"),
        ),
    )(xt)
    return jnp.transpose(yt, (1, 2, 0))


# speedup vs baseline: 3.2290x; 1.0713x over previous
import jax
import jax.numpy as jnp
import numpy as np
from jax.experimental import pallas as pl
from jax.experimental.pallas import tpu as pltpu

_ANCHORS = np.array(
    [[16, 8], [23, 103], [28, 23], [56, 47], [96, 123], [157, 248]],
    dtype=np.float32,
)
_NUM_CLASSES = 80
_IMG_DIM = 512.0
_NA = 6
_ATTRS = 5 + _NUM_CLASSES
_NG = 64
_NPOS = _NG * _NG
_NCH = _NA * _ATTRS
_STRIDE = _IMG_DIM / _NG

_BB = 8
_YB = 16
_PB = _YB * _NG


def _decode_kernel(x_ref, o_ref, tr_ref):
    yc = pl.program_id(1)
    a = pl.program_id(2)

    @pl.when(a == 0)
    def _transpose_block():
        for bl in range(_BB):
            sub = x_ref[bl].reshape(_PB, _NCH)
            tr_ref[:, bl, :] = sub.T

    base = a * _ATTRS
    o_ref[:, :, :] = tr_ref[pl.ds(base, _ATTRS)]
    iota = jax.lax.broadcasted_iota(jnp.int32, (1, 1, _PB), 2)
    gx = (iota % _NG).astype(jnp.float32)
    gy = (yc * _YB + iota // _NG).astype(jnp.float32)
    aw = jnp.float32(_ANCHORS[_NA - 1, 0])
    ah = jnp.float32(_ANCHORS[_NA - 1, 1])
    for i in range(_NA - 1):
        aw = jnp.where(a == i, jnp.float32(_ANCHORS[i, 0]), aw)
        ah = jnp.where(a == i, jnp.float32(_ANCHORS[i, 1]), ah)
    r0 = tr_ref[pl.ds(base + 0, 1)]
    r1 = tr_ref[pl.ds(base + 1, 1)]
    r2 = tr_ref[pl.ds(base + 2, 1)]
    r3 = tr_ref[pl.ds(base + 3, 1)]
    r4 = tr_ref[pl.ds(base + 4, 1)]
    o_ref[0:1] = (jax.nn.sigmoid(r0) + gx) * _STRIDE
    o_ref[1:2] = (jax.nn.sigmoid(r1) + gy) * _STRIDE
    o_ref[2:3] = jnp.exp(r2) * aw
    o_ref[3:4] = jnp.exp(r3) * ah
    o_ref[4:5] = jax.nn.sigmoid(r4)


def kernel(x):
    nB = x.shape[0]
    xt = jnp.transpose(x, (0, 2, 3, 1))
    yt = pl.pallas_call(
        _decode_kernel,
        grid=(nB // _BB, _NG // _YB, _NA),
        in_specs=[
            pl.BlockSpec(
                (_BB, _YB, _NG, _NCH), lambda b8, yc, a: (b8, yc, 0, 0)
            ),
        ],
        out_specs=pl.BlockSpec(
            (_ATTRS, _BB, _PB), lambda b8, yc, a: (0, b8, a * (_NG // _YB) + yc)
        ),
        out_shape=jax.ShapeDtypeStruct((_ATTRS, nB, _NA * _NPOS), jnp.float32),
        scratch_shapes=[pltpu.VMEM((_NCH, _BB, _PB), jnp.float32)],
        compiler_params=pltpu.CompilerParams(
            dimension_semantics=("parallel", "arbitrary", "arbitrary"),
        ),
    )(xt)
    return jnp.transpose(yt, (1, 2, 0))


# speedup vs baseline: 5.9618x; 1.8463x over previous
import jax
import jax.numpy as jnp
import numpy as np
from jax.experimental import pallas as pl
from jax.experimental.pallas import tpu as pltpu

_ANCHORS = np.array(
    [[16, 8], [23, 103], [28, 23], [56, 47], [96, 123], [157, 248]],
    dtype=np.float32,
)
_NUM_CLASSES = 80
_IMG_DIM = 512.0
_NA = 6
_ATTRS = 5 + _NUM_CLASSES
_NG = 64
_NPOS = _NG * _NG
_NCH = _NA * _ATTRS
_STRIDE = _IMG_DIM / _NG

_BB = 8
_YB = 8
_PB = _YB * _NG
_NYC = _NG // _YB


def _decode_kernel(x_ref, o_ref, stage_ref, sem):
    b8 = pl.program_id(0)
    yc = pl.program_id(1)
    step = b8 * _NYC + yc
    nsteps = pl.num_programs(0) * pl.num_programs(1)
    slot = step % 2

    def _wait_slot(s):
        for a in range(_NA):
            pltpu.make_async_copy(
                stage_ref.at[s, a],
                o_ref.at[:, pl.ds(0, _BB), pl.ds(0, _PB)],
                sem.at[s, a],
            ).wait()

    @pl.when(step >= 2)
    def _drain_two_ago():
        _wait_slot(slot)

    iota = jax.lax.broadcasted_iota(jnp.int32, (1, _PB), 1)
    gx = (iota % _NG).astype(jnp.float32)
    gy = (yc * _YB + iota // _NG).astype(jnp.float32)

    for bl in range(_BB):
        sub = x_ref[bl].reshape(_PB, _NCH)
        subT = sub.T
        for a in range(_NA):
            base = a * _ATTRS
            blk = subT[base:base + _ATTRS]
            r0 = (jax.nn.sigmoid(blk[0:1]) + gx) * _STRIDE
            r1 = (jax.nn.sigmoid(blk[1:2]) + gy) * _STRIDE
            r2 = jnp.exp(blk[2:3]) * float(_ANCHORS[a, 0])
            r3 = jnp.exp(blk[3:4]) * float(_ANCHORS[a, 1])
            r4 = jax.nn.sigmoid(blk[4:5])
            full = jnp.concatenate([r0, r1, r2, r3, r4, blk[5:]], axis=0)
            stage_ref[slot, a, :, bl, :] = full

    for a in range(_NA):
        pltpu.make_async_copy(
            stage_ref.at[slot, a],
            o_ref.at[
                :,
                pl.ds(b8 * _BB, _BB),
                pl.ds(a * _NPOS + yc * _PB, _PB),
            ],
            sem.at[slot, a],
        ).start()

    @pl.when(step == nsteps - 1)
    def _drain_tail():
        _wait_slot(1 - slot)
        _wait_slot(slot)


def kernel(x):
    nB = x.shape[0]
    xt = jnp.transpose(x, (0, 2, 3, 1))
    yt = pl.pallas_call(
        _decode_kernel,
        grid=(nB // _BB, _NYC),
        in_specs=[
            pl.BlockSpec((_BB, _YB, _NG, _NCH), lambda b8, yc: (b8, yc, 0, 0)),
        ],
        out_specs=pl.BlockSpec(memory_space=pltpu.MemorySpace.HBM),
        out_shape=jax.ShapeDtypeStruct((_ATTRS, nB, _NA * _NPOS), jnp.float32),
        scratch_shapes=[
            pltpu.VMEM((2, _NA, _ATTRS, _BB, _PB), jnp.float32),
            pltpu.SemaphoreType.DMA((2, _NA)),
        ],
        compiler_params=pltpu.CompilerParams(
            dimension_semantics=("arbitrary", "arbitrary"),
        ),
    )(xt)
    return jnp.transpose(yt, (1, 2, 0))
